# baseline (speedup 1.0000x reference)
import functools

import jax
import jax.numpy as jnp
from jax import lax
from jax.experimental import pallas as pl
from jax.experimental.pallas import tpu as pltpu
from jax.experimental.pallas import tpu_sc as plsc

D_MODEL = 768
N_PATCHES = 576
N_TOT = N_PATCHES + 1
BATCH = 64

_NUM_CORES = 2
_NUM_SUBCORES = 16
_NUM_WORKERS = _NUM_CORES * _NUM_SUBCORES
_LANES = 16
_VPR = D_MODEL // _LANES

_RCHUNK = 24
_RIN = _RCHUNK + 8
_NCH = N_PATCHES // _RCHUNK
_ITEMS = _NCH * BATCH
_ITEMS_PER_W = _ITEMS // _NUM_WORKERS


def _sc_body(in_hbm, cls_hbm, pos_hbm, out_hbm,
             pos_v, cls_v, in0, in1, ot0, ot1,
             is0, is1, os0, os1):
    c_ax = lax.axis_index("c")
    s_ax = lax.axis_index("s")
    wid = c_ax * _NUM_SUBCORES + s_ax
    item0 = wid * _ITEMS_PER_W

    in_bufs = (in0, in1)
    out_bufs = (ot0, ot1)
    in_sems = (is0, is1)
    out_sems = (os0, os1)

    pltpu.sync_copy(cls_hbm, cls_v)

    def load_pos(c):
        start = pl.multiple_of(c * _RCHUNK, 8)
        pltpu.sync_copy(pos_hbm.at[pl.ds(start, _RCHUNK)], pos_v)

    load_pos(item0 // BATCH)

    def start_in(item, i):
        c = item // BATCH
        b = item % BATCH
        start = pl.multiple_of(lax.max(c * _RCHUNK - 8, 0), 8)
        pltpu.async_copy(in_hbm.at[b, pl.ds(start, _RIN)], in_bufs[i],
                         in_sems[i])

    def wait_in(i):
        pltpu.make_async_copy(in_hbm.at[0, pl.ds(0, _RIN)], in_bufs[i],
                              in_sems[i]).wait()

    def start_out(item, i):
        c = item // BATCH
        b = item % BATCH
        start = pl.multiple_of(c * _RCHUNK, 8)
        pltpu.async_copy(out_bufs[i], out_hbm.at[b, pl.ds(start, _RCHUNK)],
                         out_sems[i])

    def wait_out(i):
        pltpu.make_async_copy(out_bufs[i], out_hbm.at[0, pl.ds(0, _RCHUNK)],
                              out_sems[i]).wait()

    start_in(item0, 0)
    start_in(item0 + 1, 1)

    def g_body(g, prev_c):
        for i in range(2):
            item = item0 + g * 2 + i
            c = item // BATCH

            @pl.when(c != prev_c)
            def _():
                load_pos(c)

            wait_in(i)

            @pl.when(g > 0)
            def _():
                wait_out(i)

            roff = lax.select(c > 0, 7, -1)

            def r_body(j, cr):
                jr = lax.max(j + roff, 0)

                @plsc.parallel_loop(0, _VPR, unroll=8)
                def _(v):
                    sl = pl.ds(v * _LANES, _LANES)
                    out_bufs[i][j, sl] = in_bufs[i][jr, sl] + pos_v[j, sl]

                return cr

            lax.fori_loop(0, _RCHUNK, r_body, 0)

            @pl.when(c == 0)
            def _():
                for v in range(_VPR):
                    sl = pl.ds(v * _LANES, _LANES)
                    out_bufs[i][0, sl] = cls_v[0, sl] + pos_v[0, sl]

            start_out(item, i)

            @pl.when(g * 2 + i + 2 < _ITEMS_PER_W)
            def _():
                start_in(item + 2, i)

            prev_c = c
        return prev_c

    lax.fori_loop(0, _ITEMS_PER_W // 2, g_body, item0 // BATCH)

    wait_out(0)
    wait_out(1)

    pltpu.sync_copy(pos_hbm.at[pl.ds(N_PATCHES, 1)], pos_v.at[pl.ds(0, 1)])
    for j in range(2):
        b = wid * 2 + j
        pltpu.sync_copy(in_hbm.at[b, pl.ds(N_PATCHES - 8, 8)],
                        in_bufs[j].at[pl.ds(0, 8)])
        for v in range(_VPR):
            sl = pl.ds(v * _LANES, _LANES)
            out_bufs[j][0, sl] = in_bufs[j][7, sl] + pos_v[0, sl]
        pltpu.sync_copy(out_bufs[j].at[pl.ds(0, 1)],
                        out_hbm.at[b, pl.ds(N_PATCHES, 1)])


_sc_call = functools.partial(
    pl.kernel,
    mesh=plsc.VectorSubcoreMesh(core_axis_name="c", subcore_axis_name="s"),
    out_type=jax.ShapeDtypeStruct((BATCH, N_TOT, D_MODEL), jnp.float32),
    compiler_params=pltpu.CompilerParams(use_tc_tiling_on_sc=True),
    scratch_types=[
        pltpu.VMEM((_RCHUNK, D_MODEL), jnp.float32),
        pltpu.VMEM((1, D_MODEL), jnp.float32),
        pltpu.VMEM((_RIN, D_MODEL), jnp.float32),
        pltpu.VMEM((_RIN, D_MODEL), jnp.float32),
        pltpu.VMEM((_RCHUNK, D_MODEL), jnp.float32),
        pltpu.VMEM((_RCHUNK, D_MODEL), jnp.float32),
        pltpu.SemaphoreType.DMA,
        pltpu.SemaphoreType.DMA,
        pltpu.SemaphoreType.DMA,
        pltpu.SemaphoreType.DMA,
    ],
)(_sc_body)


def kernel(inputs, class_embed, pos_table):
    cls = class_embed.reshape(1, D_MODEL)
    out = _sc_call(inputs, cls, pos_table)
    return out


# speedup vs baseline: 1.9396x; 1.9396x over previous
import functools

import jax
import jax.numpy as jnp
from jax import lax
from jax.experimental import pallas as pl
from jax.experimental.pallas import tpu as pltpu
from jax.experimental.pallas import tpu_sc as plsc

D_MODEL = 768
N_PATCHES = 576
N_TOT = N_PATCHES + 1
BATCH = 64

_NUM_CORES = 2
_NUM_SUBCORES = 16
_NUM_WORKERS = _NUM_CORES * _NUM_SUBCORES
_LANES = 16
_VPR = D_MODEL // _LANES

_HB = BATCH // 2
_T_PER_W = N_PATCHES // _NUM_WORKERS
_ITEMS_PER_W = 2 * _T_PER_W
_POS_ROWS = 32
_POS_PAD = N_TOT + 7


def _sc_body(in_hbm, cls_hbm, pos_hbm, out_hbm,
             pos_v, cls_v, ix0, ix1, in0, in1, ot0, ot1,
             is0, is1, os0, os1):
    c_ax = lax.axis_index("c")
    s_ax = lax.axis_index("s")
    wid = c_ax * _NUM_SUBCORES + s_ax
    t0 = wid * _T_PER_W
    a0 = pl.multiple_of((t0 // 8) * 8, 8)
    toff = t0 - a0

    in_bufs = (in0, in1)
    out_bufs = (ot0, ot1)
    ix_bufs = (ix0, ix1)
    in_sems = (is0, is1)
    out_sems = (os0, os1)

    iota = lax.iota(jnp.int32, 16)
    k_lo = iota * N_PATCHES
    k_hi = (iota + 16) * N_PATCHES

    pltpu.sync_copy(cls_hbm, cls_v)
    pltpu.sync_copy(pos_hbm.at[pl.ds(a0, _POS_ROWS)], pos_v)

    def start_in(m, i):
        t = t0 + m // 2
        bh = m % 2
        base = bh * (_HB * N_PATCHES) + t - 1
        ix_bufs[i][pl.ds(0, _LANES)] = jnp.maximum(base + k_lo, 0)
        ix_bufs[i][pl.ds(_LANES, _LANES)] = base + k_hi
        pltpu.async_copy(in_hbm.at[ix_bufs[i]], in_bufs[i], in_sems[i])

    def wait_in(i):
        pltpu.make_async_copy(in_hbm.at[ix_bufs[i]], in_bufs[i],
                              in_sems[i]).wait()

    def start_out(m, i):
        t = t0 + m // 2
        bh = m % 2
        pltpu.async_copy(out_bufs[i], out_hbm.at[t, pl.ds(bh * _HB, _HB)],
                         out_sems[i])

    def wait_out(i):
        pltpu.make_async_copy(out_bufs[i], out_hbm.at[0, pl.ds(0, _HB)],
                              out_sems[i]).wait()

    start_in(0, 0)
    start_in(1, 1)

    def g_body(g, carry):
        for i in range(2):
            m = g * 2 + i
            t = t0 + m // 2
            tr = toff + m // 2

            wait_in(i)

            @pl.when(g > 0)
            def _():
                wait_out(i)

            @pl.when(t > 0)
            def _():
                def v_body(v, cr):
                    slv = pl.ds(v * _LANES, _LANES)
                    pv = pos_v[tr, slv]

                    @plsc.parallel_loop(0, _HB, unroll=8)
                    def _(r):
                        out_bufs[i][r, slv] = in_bufs[i][r, slv] + pv

                    return cr

                lax.fori_loop(0, _VPR, v_body, 0)

            @pl.when(t == 0)
            def _():
                def v_body(v, cr):
                    slv = pl.ds(v * _LANES, _LANES)
                    cv = cls_v[0, slv] + pos_v[tr, slv]

                    @plsc.parallel_loop(0, _HB, unroll=8)
                    def _(r):
                        out_bufs[i][r, slv] = cv

                    return cr

                lax.fori_loop(0, _VPR, v_body, 0)

            start_out(m, i)

            @pl.when(m + 2 < _ITEMS_PER_W)
            def _():
                start_in(m + 2, i)

        return carry

    lax.fori_loop(0, _ITEMS_PER_W // 2, g_body, 0)

    wait_out(0)
    wait_out(1)

    @pl.when(wid >= _NUM_WORKERS - 2)
    def _():
        bh = wid - (_NUM_WORKERS - 2)
        base = bh * (_HB * N_PATCHES) + N_PATCHES - 1
        ix0[pl.ds(0, _LANES)] = base + k_lo
        ix0[pl.ds(_LANES, _LANES)] = base + k_hi
        pltpu.async_copy(in_hbm.at[ix0], in0, is0).wait()
        pltpu.sync_copy(pos_hbm.at[pl.ds(N_PATCHES, 1)], cls_v)

        def v_body(v, cr):
            slv = pl.ds(v * _LANES, _LANES)
            pv = cls_v[0, slv]

            @plsc.parallel_loop(0, _HB, unroll=8)
            def _(r):
                ot0[r, slv] = in0[r, slv] + pv

            return cr

        lax.fori_loop(0, _VPR, v_body, 0)
        pltpu.sync_copy(ot0, out_hbm.at[N_PATCHES, pl.ds(bh * _HB, _HB)])


_sc_call = functools.partial(
    pl.kernel,
    mesh=plsc.VectorSubcoreMesh(core_axis_name="c", subcore_axis_name="s"),
    out_type=jax.ShapeDtypeStruct((N_TOT, BATCH, D_MODEL), jnp.float32),
    scratch_types=[
        pltpu.VMEM((_POS_ROWS, D_MODEL), jnp.float32),
        pltpu.VMEM((1, D_MODEL), jnp.float32),
        pltpu.VMEM((_HB,), jnp.int32),
        pltpu.VMEM((_HB,), jnp.int32),
        pltpu.VMEM((_HB, D_MODEL), jnp.float32),
        pltpu.VMEM((_HB, D_MODEL), jnp.float32),
        pltpu.VMEM((_HB, D_MODEL), jnp.float32),
        pltpu.VMEM((_HB, D_MODEL), jnp.float32),
        pltpu.SemaphoreType.DMA,
        pltpu.SemaphoreType.DMA,
        pltpu.SemaphoreType.DMA,
        pltpu.SemaphoreType.DMA,
    ],
)(_sc_body)


def kernel(inputs, class_embed, pos_table):
    flat_in = inputs.reshape(BATCH * N_PATCHES, D_MODEL)
    cls = class_embed.reshape(1, D_MODEL)
    pos_pad = jnp.pad(pos_table, ((0, _POS_PAD - N_TOT), (0, 0)))
    out_t = _sc_call(flat_in, cls, pos_pad)
    return out_t.transpose(1, 0, 2)


# speedup vs baseline: 1.9521x; 1.0064x over previous
import functools

import jax
import jax.numpy as jnp
from jax import lax
from jax.experimental import pallas as pl
from jax.experimental.pallas import tpu as pltpu
from jax.experimental.pallas import tpu_sc as plsc

D_MODEL = 768
N_PATCHES = 576
N_TOT = N_PATCHES + 1
BATCH = 64

_NUM_CORES = 2
_NUM_SUBCORES = 16
_NUM_WORKERS = _NUM_CORES * _NUM_SUBCORES
_LANES = 16
_VPR = D_MODEL // _LANES

_HB = BATCH // 2
_T_PER_W = N_PATCHES // _NUM_WORKERS
_ITEMS_PER_W = 2 * _T_PER_W
_POS_ROWS = 24


def _sc_body(in_hbm, cls_hbm, pos_hbm, out_hbm,
             pos_v, cls_v, ix0, ix1, in0, in1, ot0, ot1,
             is0, is1, os0, os1):
    c_ax = lax.axis_index("c")
    s_ax = lax.axis_index("s")
    wid = c_ax * _NUM_SUBCORES + s_ax
    t0 = wid * _T_PER_W

    in_bufs = (in0, in1)
    out_bufs = (ot0, ot1)
    ix_bufs = (ix0, ix1)
    in_sems = (is0, is1)
    out_sems = (os0, os1)

    iota = lax.iota(jnp.int32, 16)
    k_lo = iota * N_PATCHES
    k_hi = (iota + 16) * N_PATCHES

    pltpu.sync_copy(cls_hbm, cls_v)
    ix0[pl.ds(0, _LANES)] = jnp.minimum(t0 + iota, N_PATCHES)
    ix0[pl.ds(8, _LANES)] = jnp.minimum(t0 + 8 + iota, N_PATCHES)
    pltpu.async_copy(pos_hbm.at[ix0.at[pl.ds(0, _POS_ROWS)]], pos_v,
                     is0).wait()

    def start_in(m, i):
        t = t0 + m // 2
        bh = m % 2
        base = bh * (_HB * N_PATCHES) + t - 1
        ix_bufs[i][pl.ds(0, _LANES)] = jnp.maximum(base + k_lo, 0)
        ix_bufs[i][pl.ds(_LANES, _LANES)] = base + k_hi
        pltpu.async_copy(in_hbm.at[ix_bufs[i]], in_bufs[i], in_sems[i])

    def wait_in(i):
        pltpu.make_async_copy(in_hbm.at[ix_bufs[i]], in_bufs[i],
                              in_sems[i]).wait()

    def start_out(m, i):
        t = t0 + m // 2
        bh = m % 2
        pltpu.async_copy(out_bufs[i], out_hbm.at[t, pl.ds(bh * _HB, _HB)],
                         out_sems[i])

    def wait_out(i):
        pltpu.make_async_copy(out_bufs[i], out_hbm.at[0, pl.ds(0, _HB)],
                              out_sems[i]).wait()

    start_in(0, 0)
    start_in(1, 1)

    def g_body(g, carry):
        for i in range(2):
            m = g * 2 + i
            t = t0 + m // 2
            tr = m // 2

            wait_in(i)

            @pl.when(g > 0)
            def _():
                wait_out(i)

            @pl.when(t > 0)
            def _():
                def v_body(v, cr):
                    slv = pl.ds(v * _LANES, _LANES)
                    pv = pos_v[tr, slv]

                    @plsc.parallel_loop(0, _HB, unroll=8)
                    def _(r):
                        out_bufs[i][r, slv] = in_bufs[i][r, slv] + pv

                    return cr

                lax.fori_loop(0, _VPR, v_body, 0)

            @pl.when(t == 0)
            def _():
                def v_body(v, cr):
                    slv = pl.ds(v * _LANES, _LANES)
                    cv = cls_v[0, slv] + pos_v[tr, slv]

                    @plsc.parallel_loop(0, _HB, unroll=8)
                    def _(r):
                        out_bufs[i][r, slv] = cv

                    return cr

                lax.fori_loop(0, _VPR, v_body, 0)

            start_out(m, i)

            @pl.when(m + 2 < _ITEMS_PER_W)
            def _():
                start_in(m + 2, i)

        return carry

    lax.fori_loop(0, _ITEMS_PER_W // 2, g_body, 0)

    wait_out(0)
    wait_out(1)

    @pl.when(wid >= _NUM_WORKERS - 2)
    def _():
        bh = wid - (_NUM_WORKERS - 2)
        base = bh * (_HB * N_PATCHES) + N_PATCHES - 1
        ix0[pl.ds(0, _LANES)] = base + k_lo
        ix0[pl.ds(_LANES, _LANES)] = base + k_hi
        pltpu.async_copy(in_hbm.at[ix0], in0, is0).wait()
        pltpu.sync_copy(pos_hbm.at[pl.ds(N_PATCHES, 1)], cls_v)

        def v_body(v, cr):
            slv = pl.ds(v * _LANES, _LANES)
            pv = cls_v[0, slv]

            @plsc.parallel_loop(0, _HB, unroll=8)
            def _(r):
                ot0[r, slv] = in0[r, slv] + pv

            return cr

        lax.fori_loop(0, _VPR, v_body, 0)
        pltpu.sync_copy(ot0, out_hbm.at[N_PATCHES, pl.ds(bh * _HB, _HB)])


_sc_call = functools.partial(
    pl.kernel,
    mesh=plsc.VectorSubcoreMesh(core_axis_name="c", subcore_axis_name="s"),
    out_type=jax.ShapeDtypeStruct((N_TOT, BATCH, D_MODEL), jnp.float32),
    scratch_types=[
        pltpu.VMEM((_POS_ROWS, D_MODEL), jnp.float32),
        pltpu.VMEM((1, D_MODEL), jnp.float32),
        pltpu.VMEM((_HB,), jnp.int32),
        pltpu.VMEM((_HB,), jnp.int32),
        pltpu.VMEM((_HB, D_MODEL), jnp.float32),
        pltpu.VMEM((_HB, D_MODEL), jnp.float32),
        pltpu.VMEM((_HB, D_MODEL), jnp.float32),
        pltpu.VMEM((_HB, D_MODEL), jnp.float32),
        pltpu.SemaphoreType.DMA,
        pltpu.SemaphoreType.DMA,
        pltpu.SemaphoreType.DMA,
        pltpu.SemaphoreType.DMA,
    ],
)(_sc_body)


def kernel(inputs, class_embed, pos_table):
    flat_in = inputs.reshape(BATCH * N_PATCHES, D_MODEL)
    cls = class_embed.reshape(1, D_MODEL)
    out_t = _sc_call(flat_in, cls, pos_table)
    return out_t.transpose(1, 0, 2)
